# baseline (speedup 1.0000x reference)
import functools

import jax
import jax.numpy as jnp
from jax import lax
from jax.experimental import pallas as pl
from jax.experimental.pallas import tpu as pltpu
from jax.experimental.pallas import tpu_sc as plsc

EMBD = 32
NW = 32
CHUNK = 1024
SUB = 128


@functools.cache
def _make_kernel(B):
    b_per_w = B // NW
    n_chunks = b_per_w // CHUNK
    mesh = plsc.VectorSubcoreMesh(core_axis_name="c", subcore_axis_name="s")

    @functools.partial(
        pl.kernel,
        mesh=mesh,
        out_type=jax.ShapeDtypeStruct((B, EMBD), jnp.float32),
        scratch_types=[
            pltpu.VMEM((CHUNK,), jnp.int32),
            pltpu.VMEM((CHUNK, EMBD), jnp.float32),
            pltpu.SemaphoreType.DMA,
        ],
        compiler_params=pltpu.CompilerParams(use_tc_tiling_on_sc=False),
    )
    def emb_kernel(idx_hbm, table_hbm, out_hbm, idx_v, rows_v, sem):
        wid = lax.axis_index("s") * 2 + lax.axis_index("c")
        base = wid * b_per_w

        def chunk_body(g, carry):
            off = base + g * CHUNK
            pltpu.sync_copy(idx_hbm.at[pl.ds(off, CHUNK)], idx_v)
            copies = [
                pltpu.async_copy(
                    table_hbm.at[idx_v.at[pl.ds(j * SUB, SUB)]],
                    rows_v.at[pl.ds(j * SUB, SUB)],
                    sem,
                )
                for j in range(CHUNK // SUB)
            ]
            for c in copies:
                c.wait()

            def relu_body(r, rcarry):
                rows_v[r, pl.ds(0, 16)] = jnp.maximum(rows_v[r, pl.ds(0, 16)], 0.0)
                rows_v[r, pl.ds(16, 16)] = jnp.maximum(rows_v[r, pl.ds(16, 16)], 0.0)
                return rcarry

            lax.fori_loop(0, CHUNK, relu_body, 0)
            pltpu.sync_copy(rows_v, out_hbm.at[pl.ds(off, CHUNK)])
            return carry

        lax.fori_loop(0, n_chunks, chunk_body, 0)

    return emb_kernel


def kernel(x, table):
    B = x.shape[0] * x.shape[1]
    flat = x.reshape(B)
    out = _make_kernel(B)(flat, table)
    return out.reshape(x.shape[0], x.shape[1], EMBD)


# speedup vs baseline: 1.0743x; 1.0743x over previous
import functools

import jax
import jax.numpy as jnp
from jax import lax
from jax.experimental import pallas as pl
from jax.experimental.pallas import tpu as pltpu
from jax.experimental.pallas import tpu_sc as plsc

EMBD = 32
NW = 32
CHUNK = 1280
SUB = 128
K = CHUNK // SUB
RELU_UNROLL = 8


@functools.cache
def _make_kernel(B):
    b_per_w = B // NW
    n_chunks = b_per_w // CHUNK
    assert n_chunks % 2 == 0 and n_chunks * CHUNK == b_per_w
    pairs = n_chunks // 2
    mesh = plsc.VectorSubcoreMesh(core_axis_name="c", subcore_axis_name="s")

    @functools.partial(
        pl.kernel,
        mesh=mesh,
        out_type=jax.ShapeDtypeStruct((B, EMBD), jnp.float32),
        scratch_types=[
            pltpu.VMEM((CHUNK,), jnp.int32),
            pltpu.VMEM((CHUNK,), jnp.int32),
            pltpu.VMEM((CHUNK, EMBD), jnp.float32),
            pltpu.VMEM((CHUNK, EMBD), jnp.float32),
            pltpu.SemaphoreType.DMA,
            pltpu.SemaphoreType.DMA,
            pltpu.SemaphoreType.DMA,
            pltpu.SemaphoreType.DMA,
        ],
        compiler_params=pltpu.CompilerParams(use_tc_tiling_on_sc=False),
    )
    def emb_kernel(idx_hbm, table_hbm, out_hbm, idx_a, idx_b, rows_a, rows_b,
                   gsem_a, gsem_b, wsem_a, wsem_b):
        wid = lax.axis_index("s") * 2 + lax.axis_index("c")
        base = wid * b_per_w

        def gather(idx_v, rows_v, sem):
            for j in range(K):
                pltpu.make_async_copy(
                    table_hbm.at[idx_v.at[pl.ds(j * SUB, SUB)]],
                    rows_v.at[pl.ds(j * SUB, SUB)],
                    sem,
                ).start()

        def gather_wait(idx_v, rows_v, sem):
            for j in range(K):
                pltpu.make_async_copy(
                    table_hbm.at[idx_v.at[pl.ds(j * SUB, SUB)]],
                    rows_v.at[pl.ds(j * SUB, SUB)],
                    sem,
                ).wait()

        def relu(rows_v):
            def body(i, carry):
                r = i * RELU_UNROLL
                for u in range(RELU_UNROLL):
                    rows_v[r + u, pl.ds(0, 16)] = jnp.maximum(
                        rows_v[r + u, pl.ds(0, 16)], 0.0)
                    rows_v[r + u, pl.ds(16, 16)] = jnp.maximum(
                        rows_v[r + u, pl.ds(16, 16)], 0.0)
                return carry
            lax.fori_loop(0, CHUNK // RELU_UNROLL, body, 0)

        def write_start(rows_v, off, sem):
            pltpu.make_async_copy(
                rows_v, out_hbm.at[pl.ds(off, CHUNK)], sem).start()

        def write_wait(rows_v, off, sem):
            pltpu.make_async_copy(
                rows_v, out_hbm.at[pl.ds(off, CHUNK)], sem).wait()

        pltpu.sync_copy(idx_hbm.at[pl.ds(base, CHUNK)], idx_a)
        gather(idx_a, rows_a, gsem_a)

        def pair_body(p, carry):
            e_off = base + (2 * p) * CHUNK
            o_off = e_off + CHUNK

            pltpu.sync_copy(idx_hbm.at[pl.ds(o_off, CHUNK)], idx_b)

            @pl.when(p > 0)
            def _():
                write_wait(rows_b, o_off - 2 * CHUNK, wsem_b)

            gather(idx_b, rows_b, gsem_b)

            gather_wait(idx_a, rows_a, gsem_a)
            relu(rows_a)
            write_start(rows_a, e_off, wsem_a)

            gather_wait(idx_b, rows_b, gsem_b)

            @pl.when(p < pairs - 1)
            def _():
                pltpu.sync_copy(idx_hbm.at[pl.ds(e_off + 2 * CHUNK, CHUNK)],
                                idx_a)
                write_wait(rows_a, e_off, wsem_a)
                gather(idx_a, rows_a, gsem_a)

            relu(rows_b)
            write_start(rows_b, o_off, wsem_b)
            return carry

        lax.fori_loop(0, pairs, pair_body, 0)

        last = base + (n_chunks - 1) * CHUNK
        write_wait(rows_a, last - CHUNK, wsem_a)
        write_wait(rows_b, last, wsem_b)

    return emb_kernel


def kernel(x, table):
    B = x.shape[0] * x.shape[1]
    flat = x.reshape(B)
    out = _make_kernel(B)(flat, table)
    return out.reshape(x.shape[0], x.shape[1], EMBD)


# speedup vs baseline: 1.4863x; 1.3835x over previous
import functools

import jax
import jax.numpy as jnp
from jax import lax
from jax.experimental import pallas as pl
from jax.experimental.pallas import tpu as pltpu
from jax.experimental.pallas import tpu_sc as plsc

VOCAB = 1000000
EMBD = 32
NW = 32
BLK = 128
HIST = 50


@functools.cache
def _make_kernel(batch):
    n_blk = batch // BLK
    blk_per_w = n_blk // NW
    pairs = HIST // 2
    mesh = plsc.VectorSubcoreMesh(core_axis_name="c", subcore_axis_name="s")

    @functools.partial(
        pl.kernel,
        mesh=mesh,
        out_type=jax.ShapeDtypeStruct((HIST, EMBD // 8, n_blk, 8, BLK),
                                      jnp.float32),
        scratch_types=[
            pltpu.VMEM((BLK * HIST,), jnp.int32),
            pltpu.VMEM((HIST, BLK), jnp.int32),
            pltpu.VMEM((BLK, EMBD), jnp.float32),
            pltpu.VMEM((BLK, EMBD), jnp.float32),
            pltpu.VMEM((EMBD, BLK), jnp.float32),
            pltpu.VMEM((EMBD, BLK), jnp.float32),
            pltpu.SemaphoreType.DMA,
            pltpu.SemaphoreType.DMA,
            pltpu.SemaphoreType.DMA,
            pltpu.SemaphoreType.DMA,
        ],
        compiler_params=pltpu.CompilerParams(use_tc_tiling_on_sc=False,
                                             needs_layout_passes=False),
    )
    def emb_kernel(idx_hbm, table_hbm, out_hbm, idx_raw, idx_t, rows_a,
                   rows_b, out_a, out_b, gsem_a, gsem_b, wsem_a, wsem_b):
        wid = lax.axis_index("s") * 2 + lax.axis_index("c")
        iota = lax.iota(jnp.int32, 16)

        def gather_start(h, rows_v, sem):
            pltpu.make_async_copy(
                table_hbm.at[idx_t.at[h]], rows_v, sem).start()

        def gather_wait(h, rows_v, sem):
            pltpu.make_async_copy(
                table_hbm.at[idx_t.at[h]], rows_v, sem).wait()

        def transpose_relu(rows_v, out_v):
            def cg_body(g8, carry):
                for c4 in range(4):
                    c = g8 * 4 + c4
                    col_ids = jnp.full((16,), 0, jnp.int32) + c
                    for q in range(BLK // 16):
                        vals = plsc.load_gather(
                            rows_v, [iota + (16 * q), col_ids])
                        out_v[c, pl.ds(16 * q, 16)] = jnp.maximum(vals, 0.0)
                return carry

            lax.fori_loop(0, EMBD // 4, cg_body, 0)

        def write_start(h, jblk, out_v, sem):
            for g in range(EMBD // 8):
                pltpu.make_async_copy(
                    out_v.at[pl.ds(8 * g, 8), :],
                    out_hbm.at[h, g, jblk], sem).start()

        def write_wait(h, jblk, out_v, sem):
            for g in range(EMBD // 8):
                pltpu.make_async_copy(
                    out_v.at[pl.ds(8 * g, 8), :],
                    out_hbm.at[h, g, jblk], sem).wait()

        def jj_body(jj, jcarry):
            jblk = wid * blk_per_w + jj

            pltpu.sync_copy(idx_hbm.at[pl.ds(jblk * BLK * HIST, BLK * HIST)],
                            idx_raw)
            for q in range(BLK // 16):
                base_ids = iota * HIST + (16 * HIST * q)

                def idxt_body(h, carry):
                    ids = plsc.load_gather(idx_raw, [base_ids + h])
                    idx_t[h, pl.ds(16 * q, 16)] = ids
                    return carry

                lax.fori_loop(0, HIST, idxt_body, 0)

            gather_start(0, rows_a, gsem_a)

            def pair_body(p, carry):
                h_e = 2 * p
                h_o = h_e + 1

                @pl.when(p > 0)
                def _():
                    write_wait(h_o - 2, jblk, out_b, wsem_b)

                gather_start(h_o, rows_b, gsem_b)

                gather_wait(h_e, rows_a, gsem_a)
                transpose_relu(rows_a, out_a)
                write_start(h_e, jblk, out_a, wsem_a)

                gather_wait(h_o, rows_b, gsem_b)

                @pl.when(p < pairs - 1)
                def _():
                    write_wait(h_e, jblk, out_a, wsem_a)
                    gather_start(h_e + 2, rows_a, gsem_a)

                transpose_relu(rows_b, out_b)
                write_start(h_o, jblk, out_b, wsem_b)
                return carry

            lax.fori_loop(0, pairs, pair_body, 0)

            write_wait(HIST - 2, jblk, out_a, wsem_a)
            write_wait(HIST - 1, jblk, out_b, wsem_b)
            return jcarry

        lax.fori_loop(0, blk_per_w, jj_body, 0)

    return emb_kernel


def kernel(x, table):
    batch, hist = x.shape
    flat = x.reshape(batch * hist)
    out5 = _make_kernel(batch)(flat, table)
    return jnp.transpose(out5, (2, 4, 0, 1, 3)).reshape(batch, hist, EMBD)
